# Initial kernel scaffold; baseline (speedup 1.0000x reference)
import jax
import jax.numpy as jnp
from jax.experimental import pallas as pl


def kernel(x, edge_index, edge_attr, batch, pre_W, pre_b, e_W1, e_b1, p_W1, p_b1, q_W1, q_b1, l_W1, l_b1, e_W2, e_b2, p_W2, p_b2, q_W2, q_b2, l_W2, l_b2, post_W, post_b, out_W, out_b):
    raise NotImplementedError("write your pallas kernel here")


# speedup vs baseline: 1.1495x; 1.1495x over previous
import numpy as np
import jax
import jax.numpy as jnp
from jax.experimental import pallas as pl
from jax.experimental.pallas import tpu as pltpu

_DELTA = float(np.log(33.0))
_N = 10000
_E = 320000
_G = 64
_EBLK = 5000
_NBLK = 2000


def _dot(a, b):
    return jnp.dot(a.astype(jnp.bfloat16), b.astype(jnp.bfloat16),
                   preferred_element_type=jnp.float32)


def _dot_hi(a, b):
    return jnp.dot(a, b, preferred_element_type=jnp.float32,
                   precision=jax.lax.Precision.HIGHEST)


def _full(shape):
    return pl.BlockSpec(shape, lambda i: tuple(0 for _ in shape))


def _rows(shape):
    return pl.BlockSpec(shape, lambda i: (i,) + tuple(0 for _ in shape[1:]))


def _pre1_body(x_ref, pw_ref, pb_ref, o_ref, s_ref, q_ref):
    o = _dot(x_ref[...], pw_ref[...]) + pb_ref[...]
    o_ref[...] = o

    @pl.when(pl.program_id(0) == 0)
    def _():
        s_ref[...] = jnp.zeros_like(s_ref)
        q_ref[...] = jnp.zeros_like(q_ref)

    s_ref[...] += jnp.sum(o, axis=0, keepdims=True)
    q_ref[...] += jnp.sum(o * o, axis=0, keepdims=True)


def _run_pre1(x, pre_W, pre_b):
    return pl.pallas_call(
        _pre1_body,
        grid=(_N // _NBLK,),
        in_specs=[_rows((_NBLK, 128)), _full((128, 64)), _full((1, 64))],
        out_specs=[_rows((_NBLK, 64)), _full((1, 64)), _full((1, 64))],
        out_shape=[jax.ShapeDtypeStruct((_N, 64), jnp.float32),
                   jax.ShapeDtypeStruct((1, 64), jnp.float32),
                   jax.ShapeDtypeStruct((1, 64), jnp.float32)],
    )(x, pre_W, pre_b.reshape(1, 64))


def _norm_body(o_ref, s_ref, q_ref, pwd_ref, pws_ref, h_ref, a_ref, b_ref):
    mu = s_ref[...] / _N
    var = q_ref[...] / _N - mu * mu
    h = jnp.maximum((o_ref[...] - mu) / jnp.sqrt(var + 1e-5), 0.0)
    h_ref[...] = h
    a_ref[...] = _dot(h, pwd_ref[...])
    b_ref[...] = _dot(h, pws_ref[...])


def _run_norm(o, s, q, pWd, pWs):
    return pl.pallas_call(
        _norm_body,
        grid=(_N // _NBLK,),
        in_specs=[_rows((_NBLK, 64)), _full((1, 64)), _full((1, 64)),
                  _full((64, 64)), _full((64, 64))],
        out_specs=[_rows((_NBLK, 64))] * 3,
        out_shape=[jax.ShapeDtypeStruct((_N, 64), jnp.float32)] * 3,
    )(o, s, q, pWd, pWs)


def _c_body(ea_ref, w1_ref, eb1_ref, pe1_ref, b1_ref,
            w2_ref, eb2_ref, pe2_ref, b2_ref, c1_ref, c2_ref):
    ea = ea_ref[...]
    e1 = _dot(ea, w1_ref[...]) + eb1_ref[...]
    c1_ref[...] = _dot(e1, pe1_ref[...]) + b1_ref[...]
    e2 = _dot(ea, w2_ref[...]) + eb2_ref[...]
    c2_ref[...] = _dot(e2, pe2_ref[...]) + b2_ref[...]


def _run_c(edge_attr, eW1, eb1, pWe1, pb1, eW2, eb2, pWe2, pb2):
    return pl.pallas_call(
        _c_body,
        grid=(_E // _EBLK,),
        in_specs=[_rows((_EBLK, 16)), _full((16, 64)), _full((1, 64)),
                  _full((64, 64)), _full((1, 64)), _full((16, 64)),
                  _full((1, 64)), _full((64, 64)), _full((1, 64))],
        out_specs=[_rows((_EBLK, 64))] * 2,
        out_shape=[jax.ShapeDtypeStruct((_E, 64), jnp.float32)] * 2,
    )(edge_attr, eW1, eb1, pWe1, pb1, eW2, eb2, pWe2, pb2)


def _mix_body(cnt_ref, usum_ref, usq_ref, umx_ref, umn_ref, a_ref, h_ref,
              qw_ref, qbias_ref, lw_ref, lb_ref, o_ref, s_ref, q_ref):
    cnt = cnt_ref[...]
    safe = jnp.maximum(cnt, 1.0)
    pos = cnt > 0
    mu = usum_ref[...] / safe
    msq = usq_ref[...] / safe
    a = a_ref[...]
    mean = jnp.where(pos, a + mu, 0.0)
    mx = jnp.where(pos, a + umx_ref[...], 0.0)
    mn = jnp.where(pos, a + umn_ref[...], 0.0)
    std = jnp.sqrt(jnp.maximum(msq - mu * mu, 0.0) + 1e-5)
    agg = jnp.concatenate([mean, mn, mx, std], axis=1)
    sl = jnp.log(cnt + 1.0)
    amp = sl / _DELTA
    att = _DELTA / jnp.where(sl > 0, sl, 1.0)
    cat = jnp.concatenate([h_ref[...], agg, agg * amp, agg * att], axis=1)
    out = _dot(cat, qw_ref[...]) + qbias_ref[...]
    out = _dot(out, lw_ref[...]) + lb_ref[...]
    o_ref[...] = out

    @pl.when(pl.program_id(0) == 0)
    def _():
        s_ref[...] = jnp.zeros_like(s_ref)
        q_ref[...] = jnp.zeros_like(q_ref)

    s_ref[...] += jnp.sum(out, axis=0, keepdims=True)
    q_ref[...] += jnp.sum(out * out, axis=0, keepdims=True)


def _run_mix(cnt, stats, a, h, qW, qbias, lW, lb):
    return pl.pallas_call(
        _mix_body,
        grid=(_N // _NBLK,),
        in_specs=[_rows((_NBLK, 1))] + [_rows((_NBLK, 64))] * 6
        + [_full((832, 64)), _full((1, 64)), _full((64, 64)),
           _full((1, 64))],
        out_specs=[_rows((_NBLK, 64)), _full((1, 64)), _full((1, 64))],
        out_shape=[jax.ShapeDtypeStruct((_N, 64), jnp.float32),
                   jax.ShapeDtypeStruct((1, 64), jnp.float32),
                   jax.ShapeDtypeStruct((1, 64), jnp.float32)],
    )(cnt, *stats, a, h, qW, qbias.reshape(1, 64), lW, lb.reshape(1, 64))


def _fin_body(o_ref, s_ref, q_ref, batch_ref, postw_ref, postb_ref,
              ow_ref, ob_ref, res_ref):
    mu = s_ref[...] / _N
    var = q_ref[...] / _N - mu * mu
    h2 = jnp.maximum((o_ref[...] - mu) / jnp.sqrt(var + 1e-5), 0.0)
    io = jax.lax.broadcasted_iota(jnp.int32, (_N, _G), 1)
    p = (batch_ref[...] == io).astype(jnp.float32)
    g = jax.lax.dot_general(p, h2, (((0,), (0,)), ((), ())),
                            preferred_element_type=jnp.float32,
                            precision=jax.lax.Precision.HIGHEST)
    g = _dot(g, postw_ref[...]) + postb_ref[...]
    gmu = jnp.mean(g, axis=0, keepdims=True)
    gvar = jnp.mean((g - gmu) * (g - gmu), axis=0, keepdims=True)
    g = jnp.maximum((g - gmu) / jnp.sqrt(gvar + 1e-5), 0.0)
    res_ref[...] = _dot(g, ow_ref[...]) + ob_ref[...]


def _run_fin(o, s, q, batch, post_W, post_b, out_W, out_b):
    return pl.pallas_call(
        _fin_body,
        out_shape=jax.ShapeDtypeStruct((_G, 1), jnp.float32),
    )(o, s, q, batch.reshape(_N, 1).astype(jnp.int32),
      post_W, post_b.reshape(1, 32), out_W, out_b.reshape(1, 1))


def _segment_stats(u, dst):
    usum = jax.ops.segment_sum(u, dst, num_segments=_N)
    usq = jax.ops.segment_sum(u * u, dst, num_segments=_N)
    umx = jax.ops.segment_max(u, dst, num_segments=_N)
    umn = jax.ops.segment_min(u, dst, num_segments=_N)
    return usum, usq, umx, umn


def kernel(x, edge_index, edge_attr, batch, pre_W, pre_b, e_W1, e_b1, p_W1,
           p_b1, q_W1, q_b1, l_W1, l_b1, e_W2, e_b2, p_W2, p_b2, q_W2, q_b2,
           l_W2, l_b2, post_W, post_b, out_W, out_b):
    src = edge_index[0]
    dst = edge_index[1]

    pWd1, pWs1, pWe1 = p_W1[:64], p_W1[64:128], p_W1[128:]
    pWd2, pWs2, pWe2 = p_W2[:64], p_W2[64:128], p_W2[128:]

    o0, s0, q0 = _run_pre1(x, pre_W, pre_b)
    h0, a1, b1 = _run_norm(o0, s0, q0, pWd1, pWs1)
    c1, c2 = _run_c(edge_attr, e_W1, e_b1.reshape(1, 64), pWe1,
                    p_b1.reshape(1, 64), e_W2, e_b2.reshape(1, 64), pWe2,
                    p_b2.reshape(1, 64))

    cnt = jax.ops.segment_sum(jnp.ones((_E, 1), jnp.float32), dst,
                              num_segments=_N)
    s1 = _segment_stats(b1[src] + c1, dst)
    o1, s1s, s1q = _run_mix(cnt, s1, a1, h0, q_W1, q_b1, l_W1, l_b1)
    h1, a2, b2 = _run_norm(o1, s1s, s1q, pWd2, pWs2)
    s2 = _segment_stats(b2[src] + c2, dst)
    o2, s2s, s2q = _run_mix(cnt, s2, a2, h1, q_W2, q_b2, l_W2, l_b2)
    return _run_fin(o2, s2s, s2q, batch, post_W, post_b, out_W, out_b)


# speedup vs baseline: 1.1508x; 1.0011x over previous
import functools
import numpy as np
import jax
import jax.numpy as jnp
from jax import lax
from jax.experimental import pallas as pl
from jax.experimental.pallas import tpu as pltpu
from jax.experimental.pallas import tpu_sc as plsc

_DELTA = float(np.log(33.0))
_N = 10000
_E = 320000
_G = 64
_EBLK = 3200
_NBLK = 2000
_NW = 32
_NPT = 320
_NPAD = _NW * _NPT
_CH = 128


def _dot(a, b):
    return jnp.dot(a.astype(jnp.bfloat16), b.astype(jnp.bfloat16),
                   preferred_element_type=jnp.float32)


def _dot_hi(a, b):
    return jnp.dot(a, b, preferred_element_type=jnp.float32,
                   precision=jax.lax.Precision.HIGHEST)


def _full(shape):
    return pl.BlockSpec(shape, lambda i: tuple(0 for _ in shape))


def _rows(shape):
    return pl.BlockSpec(shape, lambda i: (i,) + tuple(0 for _ in shape[1:]))


def _pre1_body(x_ref, pw_ref, pb_ref, o_ref, s_ref, q_ref):
    o = _dot(x_ref[...], pw_ref[...]) + pb_ref[...]
    o_ref[...] = o

    @pl.when(pl.program_id(0) == 0)
    def _():
        s_ref[...] = jnp.zeros_like(s_ref)
        q_ref[...] = jnp.zeros_like(q_ref)

    s_ref[...] += jnp.sum(o, axis=0, keepdims=True)
    q_ref[...] += jnp.sum(o * o, axis=0, keepdims=True)


def _run_pre1(x, pre_W, pre_b):
    return pl.pallas_call(
        _pre1_body,
        grid=(_N // _NBLK,),
        in_specs=[_rows((_NBLK, 128)), _full((128, 64)), _full((1, 64))],
        out_specs=[_rows((_NBLK, 64)), _full((1, 64)), _full((1, 64))],
        out_shape=[jax.ShapeDtypeStruct((_N, 64), jnp.float32),
                   jax.ShapeDtypeStruct((1, 64), jnp.float32),
                   jax.ShapeDtypeStruct((1, 64), jnp.float32)],
    )(x, pre_W, pre_b.reshape(1, 64))


def _norm_body(o_ref, s_ref, q_ref, pwd_ref, pws_ref, h_ref, a_ref, b_ref):
    mu = s_ref[...] / _N
    var = q_ref[...] / _N - mu * mu
    h = jnp.maximum((o_ref[...] - mu) / jnp.sqrt(var + 1e-5), 0.0)
    h_ref[...] = h
    a_ref[...] = _dot(h, pwd_ref[...])
    b_ref[...] = _dot(h, pws_ref[...])


def _run_norm(o, s, q, pWd, pWs):
    return pl.pallas_call(
        _norm_body,
        grid=(_N // _NBLK,),
        in_specs=[_rows((_NBLK, 64)), _full((1, 64)), _full((1, 64)),
                  _full((64, 64)), _full((64, 64))],
        out_specs=[_rows((_NBLK, 64))] * 3,
        out_shape=[jax.ShapeDtypeStruct((_N, 64), jnp.float32)] * 3,
    )(o, s, q, pWd, pWs)


def _c_body(ea_ref, w1_ref, eb1_ref, pe1_ref, b1_ref,
            w2_ref, eb2_ref, pe2_ref, b2_ref, c1_ref, c2_ref):
    ea = ea_ref[...]
    e1 = _dot(ea, w1_ref[...]) + eb1_ref[...]
    c1_ref[...] = _dot(e1, pe1_ref[...]) + b1_ref[...]
    e2 = _dot(ea, w2_ref[...]) + eb2_ref[...]
    c2_ref[...] = _dot(e2, pe2_ref[...]) + b2_ref[...]


def _run_c(edge_attr, eW1, eb1, pWe1, pb1, eW2, eb2, pWe2, pb2):
    return pl.pallas_call(
        _c_body,
        grid=(_E // _EBLK,),
        in_specs=[_rows((_EBLK, 16)), _full((16, 64)), _full((1, 64)),
                  _full((64, 64)), _full((1, 64)), _full((16, 64)),
                  _full((1, 64)), _full((64, 64)), _full((1, 64))],
        out_specs=[_rows((_EBLK, 64))] * 2,
        out_shape=[jax.ShapeDtypeStruct((_E, 64), jnp.float32)] * 2,
    )(edge_attr, eW1, eb1, pWe1, pb1, eW2, eb2, pWe2, pb2)


def _mix_body(cnt_ref, usum_ref, usq_ref, umx_ref, umn_ref, a_ref, h_ref,
              qw_ref, qbias_ref, lw_ref, lb_ref, o_ref, s_ref, q_ref):
    cnt = cnt_ref[...]
    safe = jnp.maximum(cnt, 1.0)
    pos = cnt > 0
    mu = usum_ref[...] / safe
    msq = usq_ref[...] / safe
    a = a_ref[...]
    mean = jnp.where(pos, a + mu, 0.0)
    mx = jnp.where(pos, a + umx_ref[...], 0.0)
    mn = jnp.where(pos, a + umn_ref[...], 0.0)
    std = jnp.sqrt(jnp.maximum(msq - mu * mu, 0.0) + 1e-5)
    agg = jnp.concatenate([mean, mn, mx, std], axis=1)
    sl = jnp.log(cnt + 1.0)
    amp = sl / _DELTA
    att = _DELTA / jnp.where(sl > 0, sl, 1.0)
    cat = jnp.concatenate([h_ref[...], agg, agg * amp, agg * att], axis=1)
    out = _dot(cat, qw_ref[...]) + qbias_ref[...]
    out = _dot(out, lw_ref[...]) + lb_ref[...]
    o_ref[...] = out

    @pl.when(pl.program_id(0) == 0)
    def _():
        s_ref[...] = jnp.zeros_like(s_ref)
        q_ref[...] = jnp.zeros_like(q_ref)

    s_ref[...] += jnp.sum(out, axis=0, keepdims=True)
    q_ref[...] += jnp.sum(out * out, axis=0, keepdims=True)


def _run_mix(cnt, stats, a, h, qW, qbias, lW, lb):
    return pl.pallas_call(
        _mix_body,
        grid=(_N // _NBLK,),
        in_specs=[_rows((_NBLK, 1))] + [_rows((_NBLK, 64))] * 6
        + [_full((832, 64)), _full((1, 64)), _full((64, 64)),
           _full((1, 64))],
        out_specs=[_rows((_NBLK, 64)), _full((1, 64)), _full((1, 64))],
        out_shape=[jax.ShapeDtypeStruct((_N, 64), jnp.float32),
                   jax.ShapeDtypeStruct((1, 64), jnp.float32),
                   jax.ShapeDtypeStruct((1, 64), jnp.float32)],
    )(cnt, *stats, a, h, qW, qbias.reshape(1, 64), lW, lb.reshape(1, 64))


def _fin_body(o_ref, s_ref, q_ref, batch_ref, postw_ref, postb_ref,
              ow_ref, ob_ref, res_ref):
    mu = s_ref[...] / _N
    var = q_ref[...] / _N - mu * mu
    h2 = jnp.maximum((o_ref[...] - mu) / jnp.sqrt(var + 1e-5), 0.0)
    io = jax.lax.broadcasted_iota(jnp.int32, (_N, _G), 1)
    p = (batch_ref[...] == io).astype(jnp.float32)
    g = jax.lax.dot_general(p, h2, (((0,), (0,)), ((), ())),
                            preferred_element_type=jnp.float32,
                            precision=jax.lax.Precision.HIGHEST)
    g = _dot(g, postw_ref[...]) + postb_ref[...]
    gmu = jnp.mean(g, axis=0, keepdims=True)
    gvar = jnp.mean((g - gmu) * (g - gmu), axis=0, keepdims=True)
    g = jnp.maximum((g - gmu) / jnp.sqrt(gvar + 1e-5), 0.0)
    res_ref[...] = _dot(g, ow_ref[...]) + ob_ref[...]


def _run_fin(o, s, q, batch, post_W, post_b, out_W, out_b):
    return pl.pallas_call(
        _fin_body,
        out_shape=jax.ShapeDtypeStruct((_G, 1), jnp.float32),
    )(o, s, q, batch.reshape(_N, 1).astype(jnp.int32),
      post_W, post_b.reshape(1, 32), out_W, out_b.reshape(1, 1))


def _segment_stats(u, dst):
    usum = jax.ops.segment_sum(u, dst, num_segments=_N)
    usq = jax.ops.segment_sum(u * u, dst, num_segments=_N)
    umx = jax.ops.segment_max(u, dst, num_segments=_N)
    umn = jax.ops.segment_min(u, dst, num_segments=_N)
    return usum, usq, umx, umn


def kernel(x, edge_index, edge_attr, batch, pre_W, pre_b, e_W1, e_b1, p_W1,
           p_b1, q_W1, q_b1, l_W1, l_b1, e_W2, e_b2, p_W2, p_b2, q_W2, q_b2,
           l_W2, l_b2, post_W, post_b, out_W, out_b):
    src = edge_index[0]
    dst = edge_index[1]

    pWd1, pWs1, pWe1 = p_W1[:64], p_W1[64:128], p_W1[128:]
    pWd2, pWs2, pWe2 = p_W2[:64], p_W2[64:128], p_W2[128:]

    o0, s0, q0 = _run_pre1(x, pre_W, pre_b)
    h0, a1, b1 = _run_norm(o0, s0, q0, pWd1, pWs1)
    c1, c2 = _run_c(edge_attr, e_W1, e_b1.reshape(1, 64), pWe1,
                    p_b1.reshape(1, 64), e_W2, e_b2.reshape(1, 64), pWe2,
                    p_b2.reshape(1, 64))

    cnt = jax.ops.segment_sum(jnp.ones((_E, 1), jnp.float32), dst,
                              num_segments=_N)
    s1 = _segment_stats(b1[src] + c1, dst)
    o1, s1s, s1q = _run_mix(cnt, s1, a1, h0, q_W1, q_b1, l_W1, l_b1)
    h1, a2, b2 = _run_norm(o1, s1s, s1q, pWd2, pWs2)
    s2 = _segment_stats(b2[src] + c2, dst)
    o2, s2s, s2q = _run_mix(cnt, s2, a2, h1, q_W2, q_b2, l_W2, l_b2)
    return _run_fin(o2, s2s, s2q, batch, post_W, post_b, out_W, out_b)
